# baseline (device time: 106257 ns/iter reference)
import jax
import jax.numpy as jnp
from jax import lax
from jax.experimental import pallas as pl
from jax.experimental.pallas import tpu as pltpu

N_DEV = 4


def kernel(x, w_mat):
    m_per, k = x.shape
    _, n = w_mat.shape
    n_per = n // N_DEV
    m_tot = m_per * N_DEV

    xb = x.astype(jnp.bfloat16)
    wb = w_mat.astype(jnp.bfloat16)

    def body(x_ref, w_ref, out_ref, y_ref, amax_ref,
             amax_send, amax_recv, a2a_send, a2a_recv):
        my = lax.axis_index("i")

        barrier = pltpu.get_barrier_semaphore()
        for d in range(1, N_DEV):
            peer = (my + d) % N_DEV
            pl.semaphore_signal(barrier, inc=1, device_id=(peer,),
                                device_id_type=pl.DeviceIdType.MESH)
        pl.semaphore_wait(barrier, N_DEV - 1)

        y = jnp.dot(x_ref[:, :], w_ref[:, :],
                    preferred_element_type=jnp.float32)
        y = jnp.maximum(y, 0.0)
        y_ref[:, :] = y

        amax_ref[0, :, :] = jnp.full((8, 128), jnp.max(y), jnp.float32)

        amax_rdmas = []
        for d in range(1, N_DEV):
            peer = (my + d) % N_DEV
            r = pltpu.make_async_remote_copy(
                src_ref=amax_ref.at[0],
                dst_ref=amax_ref.at[d],
                send_sem=amax_send.at[d],
                recv_sem=amax_recv.at[d],
                device_id=(peer,),
                device_id_type=pl.DeviceIdType.MESH,
            )
            r.start()
            amax_rdmas.append(r)
        for d in range(1, N_DEV):
            rr = pltpu.make_async_remote_copy(
                src_ref=amax_ref.at[0],
                dst_ref=amax_ref.at[d],
                send_sem=amax_send.at[d],
                recv_sem=amax_recv.at[d],
                device_id=(my,),
                device_id_type=pl.DeviceIdType.MESH,
            )
            rr.wait_recv()

        gmax = jnp.max(amax_ref[:, 0, 0])
        scale = jnp.maximum(gmax, 1e-30) / 127.0
        q = jnp.clip(jnp.round(y_ref[:, :] / scale), -127.0, 127.0)
        y_ref[:, :] = q * scale

        a2a_rdmas = []
        for d in range(1, N_DEV):
            peer = (my + d) % N_DEV
            r = pltpu.make_async_remote_copy(
                src_ref=y_ref.at[:, pl.ds(peer * n_per, n_per)],
                dst_ref=out_ref.at[pl.ds(my * m_per, m_per), :],
                send_sem=a2a_send.at[d],
                recv_sem=a2a_recv.at[d],
                device_id=(peer,),
                device_id_type=pl.DeviceIdType.MESH,
            )
            r.start()
            a2a_rdmas.append(r)

        out_ref[pl.ds(my * m_per, m_per), :] = y_ref[:, pl.ds(my * n_per, n_per)]

        for d in range(1, N_DEV):
            src = (my + N_DEV - d) % N_DEV
            rr = pltpu.make_async_remote_copy(
                src_ref=y_ref.at[:, pl.ds(0, n_per)],
                dst_ref=out_ref.at[pl.ds(src * m_per, m_per), :],
                send_sem=a2a_send.at[d],
                recv_sem=a2a_recv.at[d],
                device_id=(my,),
                device_id_type=pl.DeviceIdType.MESH,
            )
            rr.wait_recv()

        for r in amax_rdmas:
            r.wait_send()
        for r in a2a_rdmas:
            r.wait_send()

    return pl.pallas_call(
        body,
        out_shape=jax.ShapeDtypeStruct((m_tot, n_per), jnp.float32),
        in_specs=[
            pl.BlockSpec(memory_space=pltpu.VMEM),
            pl.BlockSpec(memory_space=pltpu.VMEM),
        ],
        out_specs=pl.BlockSpec(memory_space=pltpu.VMEM),
        scratch_shapes=[
            pltpu.VMEM((m_per, n), jnp.float32),
            pltpu.VMEM((N_DEV, 8, 128), jnp.float32),
            pltpu.SemaphoreType.DMA((N_DEV,)),
            pltpu.SemaphoreType.DMA((N_DEV,)),
            pltpu.SemaphoreType.DMA((N_DEV,)),
            pltpu.SemaphoreType.DMA((N_DEV,)),
        ],
        compiler_params=pltpu.CompilerParams(collective_id=0),
    )(xb, wb)
